# baseline (speedup 1.0000x reference)
import functools

import jax
import jax.numpy as jnp
import numpy as np
from jax import lax
from jax.experimental import pallas as pl
from jax.experimental.pallas import tpu as pltpu
from jax.experimental.pallas import tpu_sc as plsc

_NUM_SAMPLES = 16


def _sample_indices(t: int) -> np.ndarray:
    stop = np.float32(t - 1)
    frac = np.arange(_NUM_SAMPLES - 1, dtype=np.float32) / np.float32(
        _NUM_SAMPLES - 1
    )
    vals = np.concatenate([stop * frac, np.array([stop], np.float32)])
    vals = np.clip(vals, np.float32(0.0), stop)
    return vals.astype(np.int32)


@functools.partial(jax.jit, static_argnames=("t", "d"))
def _gather_rows(x2, t: int, d: int):
    idx = _sample_indices(t)
    info = plsc.get_sparse_core_info()
    nw = info.num_cores * info.num_subcores
    chunks_per_row = nw // _NUM_SAMPLES
    clen = d // chunks_per_row
    assert d % chunks_per_row == 0 and clen % 8 == 0

    k = 4
    cs = clen // k
    assert clen % k == 0 and cs % 8 == 0

    mesh = plsc.VectorSubcoreMesh(core_axis_name="c", subcore_axis_name="s")

    assert all(int(idx[r]) == (r * (t - 1)) // (_NUM_SAMPLES - 1)
               for r in range(_NUM_SAMPLES))

    def gather_kernel(x_hbm, out_hbm, buf0, buf1, isem0, isem1, osem0, osem1):
        wid = lax.axis_index("s") * info.num_cores + lax.axis_index("c")
        r = wid // chunks_per_row
        h = wid % chunks_per_row
        src = (r * (t - 1)) // (_NUM_SAMPLES - 1)
        base = h * clen
        bufs = (buf0, buf1)
        isems = (isem0, isem1)
        osems = (osem0, osem1)
        ind = [
            pltpu.make_async_copy(
                x_hbm.at[src, pl.ds(base + j * cs, cs)],
                bufs[j % 2],
                isems[j % 2],
            )
            for j in range(k)
        ]
        outd = [
            pltpu.make_async_copy(
                bufs[j % 2],
                out_hbm.at[r, pl.ds(base + j * cs, cs)],
                osems[j % 2],
            )
            for j in range(k)
        ]
        ind[0].start()
        for j in range(k):
            if j + 1 < k:
                if j >= 1:
                    outd[j - 1].wait()
                ind[j + 1].start()
            ind[j].wait()
            outd[j].start()
        outd[k - 2].wait()
        outd[k - 1].wait()

    gather_kernel = functools.partial(
        pl.kernel,
        mesh=mesh,
        out_type=jax.ShapeDtypeStruct((_NUM_SAMPLES, d), jnp.float32),
        scratch_types=[
            pltpu.VMEM((cs,), jnp.float32),
            pltpu.VMEM((cs,), jnp.float32),
            pltpu.SemaphoreType.DMA,
            pltpu.SemaphoreType.DMA,
            pltpu.SemaphoreType.DMA,
            pltpu.SemaphoreType.DMA,
        ],
    )(gather_kernel)

    return gather_kernel(x2)


def kernel(x):
    t, c, hh, ww = x.shape
    d = c * hh * ww
    out = _gather_rows(x.reshape(t, d), t, d)
    return out.reshape(_NUM_SAMPLES, c, hh, ww)


# speedup vs baseline: 1.0414x; 1.0414x over previous
import functools

import jax
import jax.numpy as jnp
import numpy as np
from jax import lax
from jax.experimental import pallas as pl
from jax.experimental.pallas import tpu as pltpu
from jax.experimental.pallas import tpu_sc as plsc

_NUM_SAMPLES = 16


def _sample_indices(t: int) -> np.ndarray:
    stop = np.float32(t - 1)
    frac = np.arange(_NUM_SAMPLES - 1, dtype=np.float32) / np.float32(
        _NUM_SAMPLES - 1
    )
    vals = np.concatenate([stop * frac, np.array([stop], np.float32)])
    vals = np.clip(vals, np.float32(0.0), stop)
    return vals.astype(np.int32)


def kernel(x):
    t, c, hh, ww = x.shape
    idx = _sample_indices(t)
    info = plsc.get_sparse_core_info()
    nw = info.num_cores * info.num_subcores
    halves = nw // _NUM_SAMPLES
    hrows = hh // halves
    k = 2
    piece = hrows // k
    assert hh % halves == 0 and hrows % k == 0 and piece % 8 == 0

    assert all(int(idx[r]) == (r * (t - 1)) // (_NUM_SAMPLES - 1)
               for r in range(_NUM_SAMPLES))

    mesh = plsc.VectorSubcoreMesh(core_axis_name="c", subcore_axis_name="s")

    @functools.partial(
        pl.kernel,
        mesh=mesh,
        out_type=jax.ShapeDtypeStruct((_NUM_SAMPLES, c, hh, ww), jnp.float32),
        scratch_types=[
            pltpu.VMEM((c, piece, ww), jnp.float32),
            pltpu.VMEM((c, piece, ww), jnp.float32),
            pltpu.SemaphoreType.DMA,
            pltpu.SemaphoreType.DMA,
            pltpu.SemaphoreType.DMA,
            pltpu.SemaphoreType.DMA,
        ],
    )
    def gather_kernel(x_hbm, out_hbm, buf0, buf1, isem0, isem1, osem0, osem1):
        wid = lax.axis_index("s") * info.num_cores + lax.axis_index("c")
        r = wid // halves
        h = wid % halves
        src = (r * (t - 1)) // (_NUM_SAMPLES - 1)
        base = h * hrows
        bufs = (buf0, buf1)
        isems = (isem0, isem1)
        osems = (osem0, osem1)
        ind = [
            pltpu.make_async_copy(
                x_hbm.at[src, :, pl.ds(base + j * piece, piece), :],
                bufs[j],
                isems[j],
            )
            for j in range(k)
        ]
        outd = [
            pltpu.make_async_copy(
                bufs[j],
                out_hbm.at[r, :, pl.ds(base + j * piece, piece), :],
                osems[j],
            )
            for j in range(k)
        ]
        for j in range(k):
            ind[j].start()
        for j in range(k):
            ind[j].wait()
            outd[j].start()
        for j in range(k):
            outd[j].wait()

    return gather_kernel(x)
